# baseline (speedup 1.0000x reference)
import functools

import jax
import jax.numpy as jnp
from jax import lax
from jax.experimental import pallas as pl
from jax.experimental.pallas import tpu as pltpu
from jax.experimental.pallas import tpu_sc as plsc

H = 1024
E = 8
TOPK = 2
EPS = 1e-06
N = 2048
A = N * TOPK
TM = 128
T = 40
P = T * TM
NW = 32


def _router_body(x_ref, wg_ref, bg_ref, i12_ref, w12_ref, aux_ref):
    n = x_ref.shape[0]
    logits = lax.dot_general(
        x_ref[...], wg_ref[...], (((1,), (1,)), ((), ())),
        precision=lax.Precision.DEFAULT,
        preferred_element_type=jnp.float32) + bg_ref[...][None, :]
    m = jnp.max(logits, axis=1, keepdims=True)
    ex = jnp.exp(logits - m)
    probs = ex / jnp.sum(ex, axis=1, keepdims=True)
    iota = lax.broadcasted_iota(jnp.int32, (n, E), 1)
    p1 = jnp.max(probs, axis=1, keepdims=True)
    i1 = jnp.min(jnp.where(probs == p1, iota, E), axis=1, keepdims=True)
    masked = jnp.where(iota == i1, -jnp.inf, probs)
    p2 = jnp.max(masked, axis=1, keepdims=True)
    i2 = jnp.min(jnp.where(masked == p2, iota, E), axis=1, keepdims=True)
    denom = p1 + p2 + EPS
    i12_ref[...] = jnp.concatenate([i1, i2], axis=1)
    w12_ref[...] = jnp.concatenate([p1 / denom, p2 / denom], axis=1)
    mask = ((iota == i1) | (iota == i2)).astype(jnp.float32)
    usage = jnp.mean(mask, axis=0)
    gates = jnp.mean(probs, axis=0)
    aux_ref[0, 0] = jnp.sum(usage * gates) * E


def _mm_body(eot_ref, nact_ref, xg_ref, we_ref, be_ref, pwt_ref, buf_ref):
    t = pl.program_id(0)

    @pl.when(t < nact_ref[0])
    def _():
        y = lax.dot_general(
            xg_ref[...].astype(jnp.bfloat16), we_ref[0].astype(jnp.bfloat16),
            (((1,), (1,)), ((), ())),
            preferred_element_type=jnp.float32) + be_ref[0]
        buf_ref[...] = pwt_ref[...] * y


def _sc_gather_body(x_hbm, ptok_hbm, out_hbm, idx_v, rows_v, sem):
    wid = lax.axis_index("s") * 2 + lax.axis_index("c")
    base = wid * (P // NW)
    for c in range(2):
        o = base + c * (P // NW // 2)
        pltpu.sync_copy(ptok_hbm.at[pl.ds(o, P // NW // 2)], idx_v)
        pltpu.async_copy(x_hbm.at[idx_v], rows_v, sem).wait()
        pltpu.sync_copy(rows_v, out_hbm.at[pl.ds(o, P // NW // 2)])


def _sc_combine_body(buf_hbm, pos0_hbm, pos1_hbm, out_hbm,
                     i0_v, i1_v, a_v, b_v, sem):
    wid = lax.axis_index("s") * 2 + lax.axis_index("c")
    tpw = N // NW
    ch = tpw // 2
    base = wid * tpw
    for c in range(2):
        o = base + c * ch
        pltpu.sync_copy(pos0_hbm.at[pl.ds(o, ch)], i0_v)
        pltpu.sync_copy(pos1_hbm.at[pl.ds(o, ch)], i1_v)
        pltpu.async_copy(buf_hbm.at[i0_v], a_v, sem).wait()
        pltpu.async_copy(buf_hbm.at[i1_v], b_v, sem).wait()

        def add_vec(i, _):
            r = i // (H // 16)
            k = (i % (H // 16)) * 16
            a_v[r, pl.ds(k, 16)] = a_v[r, pl.ds(k, 16)] + b_v[r, pl.ds(k, 16)]
            return 0

        lax.fori_loop(0, ch * (H // 16), add_vec, 0)
        pltpu.sync_copy(a_v, out_hbm.at[pl.ds(o, ch)])


@jax.jit
def kernel(x, Wg, bg, We, be):
    b, s, h = x.shape
    x_flat = x.reshape(-1, h)

    i12, w12, aux = pl.pallas_call(
        _router_body,
        in_specs=[
            pl.BlockSpec((N, h), lambda: (0, 0)),
            pl.BlockSpec((E, h), lambda: (0, 0)),
            pl.BlockSpec((E,), lambda: (0,)),
        ],
        out_specs=[
            pl.BlockSpec((N, TOPK), lambda: (0, 0)),
            pl.BlockSpec((N, TOPK), lambda: (0, 0)),
            pl.BlockSpec(memory_space=pltpu.SMEM),
        ],
        out_shape=[
            jax.ShapeDtypeStruct((N, TOPK), jnp.int32),
            jax.ShapeDtypeStruct((N, TOPK), jnp.float32),
            jax.ShapeDtypeStruct((1, 1), jnp.float32),
        ],
    )(x_flat, Wg, bg)

    flat_e = i12.reshape(A)
    oh = (flat_e[:, None] == jnp.arange(E)[None, :]).astype(jnp.int32)
    cum = jnp.cumsum(oh, axis=0)
    rank = jnp.take_along_axis(cum, flat_e[:, None], 1)[:, 0] - 1
    counts = cum[-1]
    tiles_pe = (counts + TM - 1) // TM
    cum_tiles = jnp.cumsum(tiles_pe)
    tile_off = cum_tiles - tiles_pe
    pos = tile_off[flat_e] * TM + rank
    ptok = jnp.zeros((P,), jnp.int32).at[pos].set(jnp.arange(A) // TOPK)
    pwt = jnp.zeros((P, 1), jnp.float32).at[pos, 0].set(w12.reshape(A))
    eot = jnp.minimum(
        jnp.sum(jnp.arange(T)[:, None] >= cum_tiles[None, :], axis=1),
        E - 1).astype(jnp.int32)
    nact = cum_tiles[-1:].astype(jnp.int32)
    pos2 = pos.reshape(N, TOPK)
    pos0 = pos2[:, 0]
    pos1 = pos2[:, 1]

    mesh = plsc.VectorSubcoreMesh(core_axis_name="c", subcore_axis_name="s")
    xg = pl.kernel(
        _sc_gather_body,
        out_type=jax.ShapeDtypeStruct((P, h), jnp.float32),
        mesh=mesh,
        scratch_types=[
            pltpu.VMEM((P // NW // 2,), jnp.int32),
            pltpu.VMEM((P // NW // 2, h), jnp.float32),
            pltpu.SemaphoreType.DMA,
        ],
    )(x_flat, ptok)

    buf = pl.pallas_call(
        _mm_body,
        grid_spec=pltpu.PrefetchScalarGridSpec(
            num_scalar_prefetch=2,
            grid=(T,),
            in_specs=[
                pl.BlockSpec((TM, h), lambda t, eot, nact: (t, 0)),
                pl.BlockSpec((1, h, h), lambda t, eot, nact: (eot[t], 0, 0)),
                pl.BlockSpec((1, 1, h), lambda t, eot, nact: (eot[t], 0, 0)),
                pl.BlockSpec((TM, 1), lambda t, eot, nact: (t, 0)),
            ],
            out_specs=pl.BlockSpec((TM, h), lambda t, eot, nact: (t, 0)),
        ),
        out_shape=jax.ShapeDtypeStruct((P, h), jnp.float32),
    )(eot, nact, xg, We, be.reshape(E, 1, h), pwt)

    out = pl.kernel(
        _sc_combine_body,
        out_type=jax.ShapeDtypeStruct((N, h), jnp.float32),
        mesh=mesh,
        scratch_types=[
            pltpu.VMEM((N // NW // 2,), jnp.int32),
            pltpu.VMEM((N // NW // 2,), jnp.int32),
            pltpu.VMEM((N // NW // 2, h), jnp.float32),
            pltpu.VMEM((N // NW // 2, h), jnp.float32),
            pltpu.SemaphoreType.DMA,
        ],
    )(buf, pos0, pos1)

    return out.reshape(b, s, h), aux[0, 0]


# speedup vs baseline: 3.5572x; 3.5572x over previous
import functools

import jax
import jax.numpy as jnp
from jax.experimental import pallas as pl
from jax.experimental.pallas import tpu as pltpu

H = 1024
E = 8
TOPK = 2
EPS = 1e-06
NC = 4


def _moe_body(x_ref, wg_ref, bg_ref, we_ref, be_ref,
              out_ref, aux_ref, wd_ref, xb_ref):
    e = pl.program_id(0)
    n = x_ref.shape[0]

    @pl.when(e == 0)
    def _router():
        xb_ref[...] = x_ref[...].astype(jnp.bfloat16)
        logits = jax.lax.dot_general(
            x_ref[...], wg_ref[...], (((1,), (1,)), ((), ())),
            precision=jax.lax.Precision.DEFAULT,
            preferred_element_type=jnp.float32) + bg_ref[...][None, :]
        m = jnp.max(logits, axis=1, keepdims=True)
        ex = jnp.exp(logits - m)
        probs = ex / jnp.sum(ex, axis=1, keepdims=True)
        iota = jax.lax.broadcasted_iota(jnp.int32, (n, E), 1)
        p1 = jnp.max(probs, axis=1, keepdims=True)
        i1 = jnp.min(jnp.where(probs == p1, iota, E), axis=1, keepdims=True)
        masked = jnp.where(iota == i1, -jnp.inf, probs)
        p2 = jnp.max(masked, axis=1, keepdims=True)
        i2 = jnp.min(jnp.where(masked == p2, iota, E), axis=1, keepdims=True)
        denom = p1 + p2 + EPS
        w1 = p1 / denom
        w2 = p2 / denom
        wd_ref[...] = (jnp.where(iota == i1, w1, 0.0)
                       + jnp.where(iota == i2, w2, 0.0))
        mask = ((iota == i1) | (iota == i2)).astype(jnp.float32)
        usage = jnp.mean(mask, axis=0)
        gates = jnp.mean(probs, axis=0)
        aux_ref[0, 0] = jnp.sum(usage * gates) * E

    web = we_ref[0].astype(jnp.bfloat16)
    cs = n // NC
    for c in range(NC):
        r = pl.ds(c * cs, cs)
        wcb = wd_ref[r, pl.ds(0, E)]
        w_col = jnp.sum(
            jnp.where(jax.lax.broadcasted_iota(jnp.int32, (cs, E), 1) == e,
                      wcb, 0.0), axis=1, keepdims=True)
        xw = (w_col.astype(jnp.bfloat16) * xb_ref[r, :])
        contrib = jax.lax.dot_general(
            xw, web, (((1,), (1,)), ((), ())),
            preferred_element_type=jnp.float32)

        @pl.when(e == 0)
        def _init():
            out_ref[r, :] = contrib

        @pl.when(e > 0)
        def _acc():
            out_ref[r, :] += contrib

    @pl.when(e == E - 1)
    def _bias():
        out_ref[...] += jax.lax.dot_general(
            wd_ref[...], be_ref[...], (((1,), (0,)), ((), ())),
            precision=jax.lax.Precision.DEFAULT,
            preferred_element_type=jnp.float32)


@jax.jit
def kernel(x, Wg, bg, We, be):
    b, s, h = x.shape
    x_flat = x.reshape(-1, h)
    n = x_flat.shape[0]

    out, aux = pl.pallas_call(
        _moe_body,
        grid=(E,),
        in_specs=[
            pl.BlockSpec((n, h), lambda e: (0, 0)),
            pl.BlockSpec((E, h), lambda e: (0, 0)),
            pl.BlockSpec((E,), lambda e: (0,)),
            pl.BlockSpec((1, h, h), lambda e: (e, 0, 0)),
            pl.BlockSpec((E, h), lambda e: (0, 0)),
        ],
        out_specs=[
            pl.BlockSpec((n, h), lambda e: (0, 0)),
            pl.BlockSpec(memory_space=pltpu.SMEM),
        ],
        out_shape=[
            jax.ShapeDtypeStruct((n, h), jnp.float32),
            jax.ShapeDtypeStruct((1, 1), jnp.float32),
        ],
        scratch_shapes=[
            pltpu.VMEM((n, E), jnp.float32),
            pltpu.VMEM((n, h), jnp.bfloat16),
        ],
    )(x_flat, Wg, bg, We, be)

    return out.reshape(b, s, h), aux[0, 0]
